# baseline (speedup 1.0000x reference)
import jax
import jax.numpy as jnp
from jax import lax
from jax.experimental import pallas as pl

B = 16
L = 8192
V = 4
PAD = 5000
LOUT = L + 2 * PAD


ROWS = 16


def _body(ids_ref, mask_ref, out_ref):
    g = pl.program_id(0)
    vio = lax.broadcasted_iota(jnp.int32, (V, L), 0)
    for r in range(ROWS):
        b = g * ROWS + r
        ids = ids_ref[b // 8, b % 8]
        mask = mask_ref[b // 8, b % 8]
        ids4 = jnp.broadcast_to(ids, (V, L))
        mask4 = jnp.broadcast_to(mask, (V, L))
        center = jnp.where(ids4 == vio, mask4, 0.0)
        out_ref[r] = jnp.zeros((V, LOUT), jnp.float32)
        out_ref[r, :, pl.ds(PAD, L)] = center


def kernel(input_ids, attention_mask):
    ids = input_ids.astype(jnp.int32).reshape(2, 8, L)
    mask = attention_mask.astype(jnp.float32).reshape(2, 8, L)
    return pl.pallas_call(
        _body,
        grid=(B // ROWS,),
        in_specs=[
            pl.BlockSpec((2, 8, L), lambda b: (0, 0, 0)),
            pl.BlockSpec((2, 8, L), lambda b: (0, 0, 0)),
        ],
        out_specs=pl.BlockSpec((ROWS, V, LOUT), lambda b: (b, 0, 0)),
        out_shape=jax.ShapeDtypeStruct((B, V, LOUT), jnp.float32),
    )(ids, mask)


# speedup vs baseline: 1.1877x; 1.1877x over previous
import jax
import jax.numpy as jnp
from jax import lax
from jax.experimental import pallas as pl

B = 16
L = 8192
V = 4
PAD = 5000
LOUT = L + 2 * PAD


ROWS = 8


def _body(ids_ref, mask_ref, out_ref):
    vio = lax.broadcasted_iota(jnp.int32, (V, L), 0)
    for r in range(ROWS):
        ids = ids_ref[0, r]
        mask = mask_ref[0, r]
        ids4 = jnp.broadcast_to(ids, (V, L))
        mask4 = jnp.broadcast_to(mask, (V, L))
        center = jnp.where(ids4 == vio, mask4, 0.0)
        out_ref[r] = jnp.zeros((V, LOUT), jnp.float32)
        out_ref[r, :, pl.ds(PAD, L)] = center


def kernel(input_ids, attention_mask):
    ids = input_ids.astype(jnp.int32).reshape(2, 8, L)
    mask = attention_mask.astype(jnp.float32).reshape(2, 8, L)
    return pl.pallas_call(
        _body,
        grid=(B // ROWS,),
        in_specs=[
            pl.BlockSpec((1, 8, L), lambda b: (b, 0, 0)),
            pl.BlockSpec((1, 8, L), lambda b: (b, 0, 0)),
        ],
        out_specs=pl.BlockSpec((ROWS, V, LOUT), lambda b: (b, 0, 0)),
        out_shape=jax.ShapeDtypeStruct((B, V, LOUT), jnp.float32),
    )(ids, mask)


# speedup vs baseline: 1.2240x; 1.0306x over previous
import jax
import jax.numpy as jnp
from jax import lax
from jax.experimental import pallas as pl

B = 16
L = 8192
V = 4
PAD = 5000
LOUT = L + 2 * PAD


ROWS = 8


def _body(ids_ref, mask_ref, out_ref):
    vio = lax.broadcasted_iota(jnp.int32, (V, L), 0)
    for r in range(ROWS):
        ids = ids_ref[0, r]
        mask = mask_ref[0, r]
        ids4 = jnp.broadcast_to(ids, (V, L))
        mask4 = jnp.broadcast_to(mask, (V, L))
        center = jnp.where(ids4 == vio, mask4, 0.0)
        zpad = jnp.zeros((V, PAD), jnp.float32)
        out_ref[r, :, pl.ds(0, PAD)] = zpad
        out_ref[r, :, pl.ds(PAD, L)] = center
        out_ref[r, :, pl.ds(PAD + L, PAD)] = zpad


def kernel(input_ids, attention_mask):
    ids = input_ids.astype(jnp.int32).reshape(2, 8, L)
    mask = attention_mask.astype(jnp.float32).reshape(2, 8, L)
    return pl.pallas_call(
        _body,
        grid=(B // ROWS,),
        in_specs=[
            pl.BlockSpec((1, 8, L), lambda b: (b, 0, 0)),
            pl.BlockSpec((1, 8, L), lambda b: (b, 0, 0)),
        ],
        out_specs=pl.BlockSpec((ROWS, V, LOUT), lambda b: (b, 0, 0)),
        out_shape=jax.ShapeDtypeStruct((B, V, LOUT), jnp.float32),
    )(ids, mask)
